# baseline (speedup 1.0000x reference)
import functools

import jax
import jax.numpy as jnp
from jax import lax
from jax.experimental import pallas as pl
from jax.experimental.pallas import tpu as pltpu
from jax.experimental.pallas import tpu_sc as plsc

EMBEDDING_DIM = 32
BATCH = 16384
NROWS = 1000000
LANES = 128
TILE_COLS = 7813
FULL_T = 7812
TAIL_I0 = FULL_T * LANES
TAIL_N = NROWS - TAIL_I0
R_STRIDE = TILE_COLS * 1024
APP0 = FULL_T * 1024
APP1 = R_STRIDE + FULL_T * 1024
DEPTH = 16


def _memcpy_kernel(info, mesh):
    @functools.partial(
        pl.kernel,
        mesh=mesh,
        out_type=jax.ShapeDtypeStruct((4, TILE_COLS, 8, LANES), jnp.float32),
        scratch_types=[pltpu.SemaphoreType.DMA],
    )
    def k1(table_t_hbm, tail_app_hbm, raw_hbm, sem):
        wid = lax.axis_index("s") * info.num_cores + lax.axis_index("c")
        nw = info.num_cores * info.num_subcores

        for r in range(4):
            def body(j, _, r=r):
                t = wid + j * nw
                @pl.when(j >= DEPTH)
                def _():
                    pltpu.make_async_copy(
                        table_t_hbm.at[pl.ds(8 * r, 8), pl.ds(0, LANES)],
                        raw_hbm.at[r].at[0],
                        sem,
                    ).wait()
                pltpu.async_copy(
                    table_t_hbm.at[pl.ds(8 * r, 8), pl.ds(t * LANES, LANES)],
                    raw_hbm.at[r].at[t],
                    sem,
                )
                return ()

            n_j = (FULL_T - wid + nw - 1) // nw
            lax.fori_loop(0, n_j, body, ())
            def drain(j, _, r=r):
                pltpu.make_async_copy(
                    table_t_hbm.at[pl.ds(8 * r, 8), pl.ds(0, LANES)],
                    raw_hbm.at[r].at[0],
                    sem,
                ).wait()
                return ()

            n_d = jnp.minimum(n_j, DEPTH)
            lax.fori_loop(0, n_d, drain, ())

        for a in range(2):
            @pl.when(wid == a)
            def _(a=a):
                pltpu.sync_copy(
                    tail_app_hbm.at[a], raw_hbm.at[a].at[FULL_T]
                )

    return k1


def _gather_kernel(info, mesh, b_per_w):
    n_vregs = b_per_w // 16

    @functools.partial(
        pl.kernel,
        mesh=mesh,
        out_type=jax.ShapeDtypeStruct((EMBEDDING_DIM, BATCH), jnp.float32),
        scratch_types=[
            pltpu.VMEM((b_per_w,), jnp.int32),
            pltpu.VMEM((EMBEDDING_DIM * b_per_w,), jnp.int32),
            pltpu.VMEM((EMBEDDING_DIM * b_per_w,), jnp.float32),
            pltpu.SemaphoreType.DMA,
        ],
        compiler_params=pltpu.CompilerParams(use_tc_tiling_on_sc=False),
    )
    def k2(x_hbm, flat_hbm, out_t_hbm, xv, offs_v, rows_v, sem):
        wid = lax.axis_index("s") * info.num_cores + lax.axis_index("c")
        base = wid * b_per_w
        pltpu.sync_copy(x_hbm.at[pl.ds(base, b_per_w)], xv)

        def offs_body(jv, _):
            xq = xv[pl.ds(jv * 16, 16)]
            bad = xq >= TAIL_I0
            tq = xq - TAIL_I0
            q = (xq >> 7) * 1024 + (xq & 127)
            bbq = (
                jnp.where(tq < 32, jnp.int32(APP0), jnp.int32(APP1 - 1024))
                + tq * EMBEDDING_DIM
            )
            for d in range(EMBEDDING_DIM):
                woff_good = q + ((d // 8) * R_STRIDE + (d % 8) * LANES)
                woff = jnp.where(bad, bbq + d, woff_good)
                offs_v[pl.ds(d * b_per_w + jv * 16, 16)] = woff
            return ()

        lax.fori_loop(0, n_vregs, offs_body, ())

        pltpu.async_copy(flat_hbm.at[offs_v], rows_v, sem).wait()

        for d in range(EMBEDDING_DIM):
            pltpu.sync_copy(
                rows_v.at[pl.ds(d * b_per_w, b_per_w)],
                out_t_hbm.at[d].at[pl.ds(base, b_per_w)],
            )

    return k2


def kernel(x, table):
    info = plsc.get_sparse_core_info()
    nw = info.num_cores * info.num_subcores
    b_per_w = BATCH // nw

    mesh = plsc.VectorSubcoreMesh(core_axis_name="c", subcore_axis_name="s")

    tail_app = jnp.reshape(
        lax.slice(table, (TAIL_I0, 0), (NROWS, EMBEDDING_DIM)), (2, 8, LANES)
    )
    raw = _memcpy_kernel(info, mesh)(table.T, tail_app)
    flat = jnp.reshape(raw, (-1,))
    out_t = _gather_kernel(info, mesh, b_per_w)(x, flat)
    return out_t.T
